# baseline (speedup 1.0000x reference)
import jax
import jax.numpy as jnp
from jax import lax
from jax.experimental import pallas as pl
from jax.experimental.pallas import tpu as pltpu
from jax.experimental.pallas import tpu_sc as plsc

N = 10000
E = 320000
D = 128
NC = 2
NS = 16
NW = NC * NS
K = 64
CHUNKS = 160
EPT = CHUNKS * K
E_PAD = EPT * NW
NPAD = 640 * NS
RPT = NPAD // NS

_MESH = plsc.VectorSubcoreMesh(core_axis_name="c", subcore_axis_name="s")


def _fill_1d(ref, n, value):
    v = jnp.full((16,), value, jnp.float32)

    def body(i, carry):
        ref[pl.ds(i * 16, 16)] = v
        return carry

    lax.fori_loop(0, n // 16, body, 0)


def _fill_2d(ref, nrows, ncols, value):
    v = jnp.full((16,), value, jnp.float32)
    nv = ncols // 16

    def body(i, carry):
        ref[i // nv, pl.ds((i % nv) * 16, 16)] = v
        return carry

    lax.fori_loop(0, nrows * nv, body, 0)


def _deg_body(cols_hbm, deg_out, cols_v, ones_v, zb_v, acc_sh):
    c = lax.axis_index("c")
    s = lax.axis_index("s")
    _fill_1d(ones_v, K, 1.0)
    _fill_1d(zb_v, 640, 0.0)
    pltpu.sync_copy(zb_v, acc_sh.at[pl.ds(s * 640, 640)])
    plsc.subcore_barrier()
    pltpu.sync_copy(cols_hbm.at[c, s], cols_v)

    def body(j, carry):
        pltpu.sync_copy(ones_v, acc_sh.at[cols_v.at[j]], add=True)
        return carry

    lax.fori_loop(0, CHUNKS, body, 0)
    plsc.subcore_barrier()
    pltpu.sync_copy(acc_sh.at[pl.ds(s * 640, 640)],
                    deg_out.at[c, pl.ds(s * 640, 640)])


_deg_call = pl.kernel(
    _deg_body,
    out_type=jax.ShapeDtypeStruct((NC, NPAD), jnp.float32),
    mesh=_MESH,
    scratch_types=[
        pltpu.VMEM((CHUNKS, K), jnp.int32),
        pltpu.VMEM((K,), jnp.float32),
        pltpu.VMEM((640,), jnp.float32),
        pltpu.VMEM_SHARED((NPAD,), jnp.float32),
    ],
)


SPAN = 32
NSPANS = CHUNKS // SPAN


def _agg_body(xw_hbm, rows_hbm, cols_hbm, out_hbm,
              idx_v, gbuf, acc_sh, gsem, ssem):
    c = lax.axis_index("c")
    s = lax.axis_index("s")
    _fill_2d(gbuf.at[0], K, D, 0.0)
    for t in range(RPT // K):
        pltpu.sync_copy(gbuf.at[0], acc_sh.at[pl.ds(s * RPT + t * K, K)])
    plsc.subcore_barrier()

    def stage(off):
        pltpu.sync_copy(rows_hbm.at[c, s, pl.ds(off, SPAN)],
                        idx_v.at[pl.ds(0, SPAN)])
        pltpu.sync_copy(cols_hbm.at[c, s, pl.ds(off, SPAN)],
                        idx_v.at[pl.ds(SPAN, SPAN)])

    def gather(j, b):
        pltpu.async_copy(xw_hbm.at[idx_v.at[j]], gbuf.at[b], gsem)

    def scatter(j, b):
        pltpu.async_copy(gbuf.at[b], acc_sh.at[idx_v.at[SPAN + j]],
                         ssem, add=True)

    def wait_g():
        pltpu.make_async_copy(xw_hbm.at[pl.ds(0, K)], gbuf.at[0], gsem).wait()

    def wait_s():
        pltpu.make_async_copy(xw_hbm.at[pl.ds(0, K)], gbuf.at[0], ssem).wait()

    def run_span(n):
        gather(0, 0)
        gather(1, 1)
        wait_g()
        scatter(0, 0)
        gather(2, 2)
        wait_g()
        scatter(1, 1)

        def body(t, carry):
            wait_s()
            gather(t + 1, lax.rem(t + 1, 3))
            wait_g()
            scatter(t, lax.rem(t, 3))
            return carry

        lax.fori_loop(2, n - 1, body, 0)
        wait_s()
        wait_g()
        scatter(n - 1, (n - 1) % 3)
        wait_s()
        wait_s()

    for sp in range(NSPANS):
        stage(sp * SPAN)
        run_span(SPAN)
    plsc.subcore_barrier()
    pltpu.sync_copy(acc_sh.at[pl.ds(s * RPT, RPT)],
                    out_hbm.at[c, pl.ds(s * RPT, RPT)])


_agg_call = pl.kernel(
    _agg_body,
    out_type=jax.ShapeDtypeStruct((NC, NPAD, D), jnp.float32),
    mesh=_MESH,
    scratch_types=[
        pltpu.VMEM((2 * SPAN, K), jnp.int32),
        pltpu.VMEM((3, K, D), jnp.float32),
        pltpu.VMEM_SHARED((NPAD, D), jnp.float32),
        pltpu.SemaphoreType.DMA,
        pltpu.SemaphoreType.DMA,
    ],
)


RB = 2048
GRID = (NPAD // RB,)


def _dis_of(deg_ref):
    deg = deg_ref[0, :] + deg_ref[1, :]
    return jnp.where(deg > 0, lax.rsqrt(deg), 0.0)


def _mm1_body(x_ref, w_ref, deg_ref, o_ref):
    dis = _dis_of(deg_ref)
    xw = jnp.dot(x_ref[...], w_ref[...],
                 preferred_element_type=jnp.float32,
                 precision=lax.Precision.HIGHEST)
    o_ref[...] = xw * dis[:, None]


_mm1_call = pl.pallas_call(
    _mm1_body,
    grid=GRID,
    in_specs=[
        pl.BlockSpec((RB, D), lambda i: (i, 0)),
        pl.BlockSpec((D, D), lambda i: (0, 0)),
        pl.BlockSpec((NC, RB), lambda i: (0, i)),
    ],
    out_specs=pl.BlockSpec((RB, D), lambda i: (i, 0)),
    out_shape=jax.ShapeDtypeStruct((N, D), jnp.float32),
)


def _mm2_body(agg_ref, deg_ref, w_ref, b_ref, o_ref):
    dis = _dis_of(deg_ref)
    agg = agg_ref[0] + agg_ref[1]
    h = jnp.maximum(agg * dis[:, None] + b_ref[...], 0.0)
    hw = jnp.dot(h, w_ref[...],
                 preferred_element_type=jnp.float32,
                 precision=lax.Precision.HIGHEST)
    o_ref[...] = hw * dis[:, None]


_mm2_call = pl.pallas_call(
    _mm2_body,
    grid=GRID,
    in_specs=[
        pl.BlockSpec((NC, RB, D), lambda i: (0, i, 0)),
        pl.BlockSpec((NC, RB), lambda i: (0, i)),
        pl.BlockSpec((D, D), lambda i: (0, 0)),
        pl.BlockSpec((1, D), lambda i: (0, 0)),
    ],
    out_specs=pl.BlockSpec((RB, D), lambda i: (i, 0)),
    out_shape=jax.ShapeDtypeStruct((N, D), jnp.float32),
)


def _final_body(agg_ref, deg_ref, b_ref, o_ref):
    dis = _dis_of(deg_ref)
    agg = agg_ref[0] + agg_ref[1]
    o_ref[...] = agg * dis[:, None] + b_ref[...]


_final_call = pl.pallas_call(
    _final_body,
    grid=GRID,
    in_specs=[
        pl.BlockSpec((NC, RB, D), lambda i: (0, i, 0)),
        pl.BlockSpec((NC, RB), lambda i: (0, i)),
        pl.BlockSpec((1, D), lambda i: (0, 0)),
    ],
    out_specs=pl.BlockSpec((RB, D), lambda i: (i, 0)),
    out_shape=jax.ShapeDtypeStruct((N, D), jnp.float32),
)


@jax.jit
def kernel(x, edgeIndex, W1, b1, W2, b2):
    P = E_PAD - E
    pad_rows = jnp.arange(P, dtype=jnp.int32) % N
    pad_cols = N + jnp.arange(P, dtype=jnp.int32) % (NPAD - N)
    rows = jnp.concatenate([edgeIndex[0], pad_rows]).reshape(NC, NS, CHUNKS, K)
    cols = jnp.concatenate([edgeIndex[1], pad_cols]).reshape(NC, NS, CHUNKS, K)
    deg2 = _deg_call(cols)
    xw1 = _mm1_call(x, W1, deg2)
    agg1 = _agg_call(xw1, rows, cols)
    xw2 = _mm2_call(agg1, deg2, W2, b1.reshape(1, D))
    agg2 = _agg_call(xw2, rows, cols)
    return _final_call(agg2, deg2, b2.reshape(1, D))


# speedup vs baseline: 1.0514x; 1.0514x over previous
import jax
import jax.numpy as jnp
from jax import lax
from jax.experimental import pallas as pl
from jax.experimental.pallas import tpu as pltpu
from jax.experimental.pallas import tpu_sc as plsc

N = 10000
E = 320000
D = 128
NC = 2
NS = 16
NW = NC * NS
K = 64
CHUNKS = 160
EPT = CHUNKS * K
E_PAD = EPT * NW
NPAD = 640 * NS
RPT = NPAD // NS

_MESH = plsc.VectorSubcoreMesh(core_axis_name="c", subcore_axis_name="s")


def _fill_1d(ref, n, value):
    v = jnp.full((16,), value, jnp.float32)

    def body(i, carry):
        ref[pl.ds(i * 16, 16)] = v
        return carry

    lax.fori_loop(0, n // 16, body, 0)


def _fill_2d(ref, nrows, ncols, value):
    v = jnp.full((16,), value, jnp.float32)
    nv = ncols // 16

    def body(i, carry):
        ref[i // nv, pl.ds((i % nv) * 16, 16)] = v
        return carry

    lax.fori_loop(0, nrows * nv, body, 0)


DEG_DEPTH = 8


def _deg_body(cols_hbm, deg_out, cols_v, ones_v, zb_v, acc_sh, dsem):
    c = lax.axis_index("c")
    s = lax.axis_index("s")
    _fill_1d(ones_v, K, 1.0)
    _fill_1d(zb_v, 640, 0.0)
    pltpu.sync_copy(zb_v, acc_sh.at[pl.ds(s * 640, 640)])
    plsc.subcore_barrier()
    pltpu.sync_copy(cols_hbm.at[c, s], cols_v)

    def issue(j):
        pltpu.async_copy(ones_v, acc_sh.at[cols_v.at[j]], dsem, add=True)

    def wait_d():
        pltpu.make_async_copy(deg_out.at[0, pl.ds(0, K)], ones_v, dsem).wait()

    for j in range(DEG_DEPTH):
        issue(j)

    def body(j, carry):
        wait_d()
        issue(j + DEG_DEPTH)
        return carry

    lax.fori_loop(0, CHUNKS - DEG_DEPTH, body, 0)
    for _ in range(DEG_DEPTH):
        wait_d()
    plsc.subcore_barrier()
    pltpu.sync_copy(acc_sh.at[pl.ds(s * 640, 640)],
                    deg_out.at[c, pl.ds(s * 640, 640)])


_deg_call = pl.kernel(
    _deg_body,
    out_type=jax.ShapeDtypeStruct((NC, NPAD), jnp.float32),
    mesh=_MESH,
    scratch_types=[
        pltpu.VMEM((CHUNKS, K), jnp.int32),
        pltpu.VMEM((K,), jnp.float32),
        pltpu.VMEM((640,), jnp.float32),
        pltpu.VMEM_SHARED((NPAD,), jnp.float32),
        pltpu.SemaphoreType.DMA,
    ],
)


SPAN = 32
NSPANS = CHUNKS // SPAN


def _agg_body(xw_hbm, rows_hbm, cols_hbm, out_hbm,
              idx_v, gbuf, acc_sh, gsem, ssem):
    c = lax.axis_index("c")
    s = lax.axis_index("s")
    _fill_2d(gbuf.at[0], K, D, 0.0)
    for t in range(RPT // K):
        pltpu.sync_copy(gbuf.at[0], acc_sh.at[pl.ds(s * RPT + t * K, K)])
    plsc.subcore_barrier()

    def stage(off, ib):
        pltpu.sync_copy(rows_hbm.at[c, s, pl.ds(off, SPAN)],
                        idx_v.at[ib, pl.ds(0, SPAN)])
        pltpu.sync_copy(cols_hbm.at[c, s, pl.ds(off, SPAN)],
                        idx_v.at[ib, pl.ds(SPAN, SPAN)])

    def gather(t, b):
        ib = lax.rem(lax.div(t, SPAN), 2)
        pltpu.async_copy(xw_hbm.at[idx_v.at[ib, lax.rem(t, SPAN)]],
                         gbuf.at[b], gsem)

    def scatter(t, b):
        ib = lax.rem(lax.div(t, SPAN), 2)
        pltpu.async_copy(gbuf.at[b],
                         acc_sh.at[idx_v.at[ib, SPAN + lax.rem(t, SPAN)]],
                         ssem, add=True)

    def wait_g():
        pltpu.make_async_copy(xw_hbm.at[pl.ds(0, K)], gbuf.at[0], gsem).wait()

    def wait_s():
        pltpu.make_async_copy(xw_hbm.at[pl.ds(0, K)], gbuf.at[0], ssem).wait()

    n = CHUNKS
    stage(0, 0)
    gather(0, 0)
    gather(1, 1)
    wait_g()
    scatter(0, 0)
    gather(2, 2)
    wait_g()
    scatter(1, 1)

    def body(t, carry):
        @pl.when(jnp.logical_and(lax.rem(t, SPAN) == 4,
                                 t < (NSPANS - 1) * SPAN))
        def _():
            sp_next = lax.div(t, SPAN) + 1
            stage(sp_next * SPAN, lax.rem(sp_next, 2))

        wait_s()
        gather(t + 1, lax.rem(t + 1, 3))
        wait_g()
        scatter(t, lax.rem(t, 3))
        return carry

    lax.fori_loop(2, n - 1, body, 0)
    wait_s()
    wait_g()
    scatter(n - 1, (n - 1) % 3)
    wait_s()
    wait_s()
    plsc.subcore_barrier()
    pltpu.sync_copy(acc_sh.at[pl.ds(s * RPT, RPT)],
                    out_hbm.at[c, pl.ds(s * RPT, RPT)])


_agg_call = pl.kernel(
    _agg_body,
    out_type=jax.ShapeDtypeStruct((NC, NPAD, D), jnp.float32),
    mesh=_MESH,
    scratch_types=[
        pltpu.VMEM((2, 2 * SPAN, K), jnp.int32),
        pltpu.VMEM((3, K, D), jnp.float32),
        pltpu.VMEM_SHARED((NPAD, D), jnp.float32),
        pltpu.SemaphoreType.DMA,
        pltpu.SemaphoreType.DMA,
    ],
)


RB = 2048
GRID = (NPAD // RB,)


def _dis_of(deg_ref):
    deg = deg_ref[0, :] + deg_ref[1, :]
    return jnp.where(deg > 0, lax.rsqrt(deg), 0.0)


def _mm1_body(x_ref, w_ref, deg_ref, o_ref):
    dis = _dis_of(deg_ref)
    xw = jnp.dot(x_ref[...], w_ref[...],
                 preferred_element_type=jnp.float32,
                 precision=lax.Precision.HIGHEST)
    o_ref[...] = xw * dis[:, None]


_mm1_call = pl.pallas_call(
    _mm1_body,
    grid=GRID,
    in_specs=[
        pl.BlockSpec((RB, D), lambda i: (i, 0)),
        pl.BlockSpec((D, D), lambda i: (0, 0)),
        pl.BlockSpec((NC, RB), lambda i: (0, i)),
    ],
    out_specs=pl.BlockSpec((RB, D), lambda i: (i, 0)),
    out_shape=jax.ShapeDtypeStruct((N, D), jnp.float32),
)


def _mm2_body(agg_ref, deg_ref, w_ref, b_ref, o_ref):
    dis = _dis_of(deg_ref)
    agg = agg_ref[0] + agg_ref[1]
    h = jnp.maximum(agg * dis[:, None] + b_ref[...], 0.0)
    hw = jnp.dot(h, w_ref[...],
                 preferred_element_type=jnp.float32,
                 precision=lax.Precision.HIGHEST)
    o_ref[...] = hw * dis[:, None]


_mm2_call = pl.pallas_call(
    _mm2_body,
    grid=GRID,
    in_specs=[
        pl.BlockSpec((NC, RB, D), lambda i: (0, i, 0)),
        pl.BlockSpec((NC, RB), lambda i: (0, i)),
        pl.BlockSpec((D, D), lambda i: (0, 0)),
        pl.BlockSpec((1, D), lambda i: (0, 0)),
    ],
    out_specs=pl.BlockSpec((RB, D), lambda i: (i, 0)),
    out_shape=jax.ShapeDtypeStruct((N, D), jnp.float32),
)


def _final_body(agg_ref, deg_ref, b_ref, o_ref):
    dis = _dis_of(deg_ref)
    agg = agg_ref[0] + agg_ref[1]
    o_ref[...] = agg * dis[:, None] + b_ref[...]


_final_call = pl.pallas_call(
    _final_body,
    grid=GRID,
    in_specs=[
        pl.BlockSpec((NC, RB, D), lambda i: (0, i, 0)),
        pl.BlockSpec((NC, RB), lambda i: (0, i)),
        pl.BlockSpec((1, D), lambda i: (0, 0)),
    ],
    out_specs=pl.BlockSpec((RB, D), lambda i: (i, 0)),
    out_shape=jax.ShapeDtypeStruct((N, D), jnp.float32),
)


@jax.jit
def kernel(x, edgeIndex, W1, b1, W2, b2):
    P = E_PAD - E
    pad_rows = jnp.arange(P, dtype=jnp.int32) % N
    pad_cols = N + jnp.arange(P, dtype=jnp.int32) % (NPAD - N)
    rows = jnp.concatenate([edgeIndex[0], pad_rows]).reshape(NC, NS, CHUNKS, K)
    cols = jnp.concatenate([edgeIndex[1], pad_cols]).reshape(NC, NS, CHUNKS, K)
    deg2 = _deg_call(cols)
    xw1 = _mm1_call(x, W1, deg2)
    agg1 = _agg_call(xw1, rows, cols)
    xw2 = _mm2_call(agg1, deg2, W2, b1.reshape(1, D))
    agg2 = _agg_call(xw2, rows, cols)
    return _final_call(agg2, deg2, b2.reshape(1, D))


# speedup vs baseline: 1.0601x; 1.0082x over previous
import jax
import jax.numpy as jnp
from jax import lax
from jax.experimental import pallas as pl
from jax.experimental.pallas import tpu as pltpu
from jax.experimental.pallas import tpu_sc as plsc

N = 10000
E = 320000
D = 128
NC = 2
NS = 16
NW = NC * NS
K = 64
CHUNKS = 160
EPT = CHUNKS * K
E_PAD = EPT * NW
NPAD = 640 * NS
RPT = NPAD // NS

_MESH = plsc.VectorSubcoreMesh(core_axis_name="c", subcore_axis_name="s")


def _fill_1d(ref, n, value):
    v = jnp.full((16,), value, jnp.float32)

    def body(i, carry):
        ref[pl.ds(i * 16, 16)] = v
        return carry

    lax.fori_loop(0, n // 16, body, 0)


def _fill_2d(ref, nrows, ncols, value):
    v = jnp.full((16,), value, jnp.float32)
    nv = ncols // 16

    def body(i, carry):
        ref[i // nv, pl.ds((i % nv) * 16, 16)] = v
        return carry

    lax.fori_loop(0, nrows * nv, body, 0)


DEG_DEPTH = 8


def _deg_body(cols_hbm, deg_out, cols_v, ones_v, zb_v, acc_sh, dsem):
    c = lax.axis_index("c")
    s = lax.axis_index("s")
    _fill_1d(ones_v, K, 1.0)
    _fill_1d(zb_v, 640, 0.0)
    pltpu.sync_copy(zb_v, acc_sh.at[pl.ds(s * 640, 640)])
    plsc.subcore_barrier()
    pltpu.sync_copy(cols_hbm.at[c, s], cols_v)

    def issue(j):
        pltpu.async_copy(ones_v, acc_sh.at[cols_v.at[j]], dsem, add=True)

    def wait_d():
        pltpu.make_async_copy(deg_out.at[0, pl.ds(0, K)], ones_v, dsem).wait()

    for j in range(DEG_DEPTH):
        issue(j)

    def body(j, carry):
        wait_d()
        issue(j + DEG_DEPTH)
        return carry

    lax.fori_loop(0, CHUNKS - DEG_DEPTH, body, 0)
    for _ in range(DEG_DEPTH):
        wait_d()
    plsc.subcore_barrier()
    pltpu.sync_copy(acc_sh.at[pl.ds(s * 640, 640)],
                    deg_out.at[c, pl.ds(s * 640, 640)])


_deg_call = pl.kernel(
    _deg_body,
    out_type=jax.ShapeDtypeStruct((NC, NPAD), jnp.float32),
    mesh=_MESH,
    scratch_types=[
        pltpu.VMEM((CHUNKS, K), jnp.int32),
        pltpu.VMEM((K,), jnp.float32),
        pltpu.VMEM((640,), jnp.float32),
        pltpu.VMEM_SHARED((NPAD,), jnp.float32),
        pltpu.SemaphoreType.DMA,
    ],
)


SPAN = 32
NSPANS = CHUNKS // SPAN


def _agg_body(xw_hbm, rows_hbm, cols_hbm, out_hbm,
              idx_v, gbuf, acc_sh, gsem, ssem):
    c = lax.axis_index("c")
    s = lax.axis_index("s")

    def stage(off, ib):
        pltpu.sync_copy(rows_hbm.at[c, s, pl.ds(off, SPAN)],
                        idx_v.at[ib, pl.ds(0, SPAN)])
        pltpu.sync_copy(cols_hbm.at[c, s, pl.ds(off, SPAN)],
                        idx_v.at[ib, pl.ds(SPAN, SPAN)])

    def gather(t, b):
        ib = lax.rem(lax.div(t, SPAN), 2)
        pltpu.async_copy(xw_hbm.at[idx_v.at[ib, lax.rem(t, SPAN)]],
                         gbuf.at[b], gsem)

    def scatter(t, b):
        ib = lax.rem(lax.div(t, SPAN), 2)
        pltpu.async_copy(gbuf.at[b],
                         acc_sh.at[idx_v.at[ib, SPAN + lax.rem(t, SPAN)]],
                         ssem, add=True)

    def wait_g():
        pltpu.make_async_copy(xw_hbm.at[pl.ds(0, K)], gbuf.at[0], gsem).wait()

    def wait_s():
        pltpu.make_async_copy(xw_hbm.at[pl.ds(0, K)], gbuf.at[0], ssem).wait()

    n = CHUNKS
    stage(0, 0)
    gather(0, 0)
    gather(1, 1)
    _fill_2d(gbuf.at[2], K, D, 0.0)
    for t in range(RPT // K):
        pltpu.sync_copy(gbuf.at[2], acc_sh.at[pl.ds(s * RPT + t * K, K)])
    plsc.subcore_barrier()
    wait_g()
    scatter(0, 0)
    gather(2, 2)
    wait_g()
    scatter(1, 1)

    def body(t, carry):
        @pl.when(jnp.logical_and(lax.rem(t, SPAN) == 4,
                                 t < (NSPANS - 1) * SPAN))
        def _():
            sp_next = lax.div(t, SPAN) + 1
            stage(sp_next * SPAN, lax.rem(sp_next, 2))

        wait_s()
        gather(t + 1, lax.rem(t + 1, 3))
        wait_g()
        scatter(t, lax.rem(t, 3))
        return carry

    lax.fori_loop(2, n - 1, body, 0)
    wait_s()
    wait_g()
    scatter(n - 1, (n - 1) % 3)
    wait_s()
    wait_s()
    plsc.subcore_barrier()
    pltpu.sync_copy(acc_sh.at[pl.ds(s * RPT, RPT)],
                    out_hbm.at[c, pl.ds(s * RPT, RPT)])


_agg_call = pl.kernel(
    _agg_body,
    out_type=jax.ShapeDtypeStruct((NC, NPAD, D), jnp.float32),
    mesh=_MESH,
    scratch_types=[
        pltpu.VMEM((2, 2 * SPAN, K), jnp.int32),
        pltpu.VMEM((3, K, D), jnp.float32),
        pltpu.VMEM_SHARED((NPAD, D), jnp.float32),
        pltpu.SemaphoreType.DMA,
        pltpu.SemaphoreType.DMA,
    ],
)


RB = 2048
GRID = (NPAD // RB,)


def _dis_of(deg_ref):
    deg = deg_ref[0, :] + deg_ref[1, :]
    return jnp.where(deg > 0, lax.rsqrt(deg), 0.0)


def _mm1_body(x_ref, w_ref, deg_ref, o_ref):
    dis = _dis_of(deg_ref)
    xw = jnp.dot(x_ref[...], w_ref[...],
                 preferred_element_type=jnp.float32,
                 precision=lax.Precision.HIGHEST)
    o_ref[...] = xw * dis[:, None]


_mm1_call = pl.pallas_call(
    _mm1_body,
    grid=GRID,
    in_specs=[
        pl.BlockSpec((RB, D), lambda i: (i, 0)),
        pl.BlockSpec((D, D), lambda i: (0, 0)),
        pl.BlockSpec((NC, RB), lambda i: (0, i)),
    ],
    out_specs=pl.BlockSpec((RB, D), lambda i: (i, 0)),
    out_shape=jax.ShapeDtypeStruct((N, D), jnp.float32),
)


def _mm2_body(agg_ref, deg_ref, w_ref, b_ref, o_ref):
    dis = _dis_of(deg_ref)
    agg = agg_ref[0] + agg_ref[1]
    h = jnp.maximum(agg * dis[:, None] + b_ref[...], 0.0)
    hw = jnp.dot(h, w_ref[...],
                 preferred_element_type=jnp.float32,
                 precision=lax.Precision.HIGHEST)
    o_ref[...] = hw * dis[:, None]


_mm2_call = pl.pallas_call(
    _mm2_body,
    grid=GRID,
    in_specs=[
        pl.BlockSpec((NC, RB, D), lambda i: (0, i, 0)),
        pl.BlockSpec((NC, RB), lambda i: (0, i)),
        pl.BlockSpec((D, D), lambda i: (0, 0)),
        pl.BlockSpec((1, D), lambda i: (0, 0)),
    ],
    out_specs=pl.BlockSpec((RB, D), lambda i: (i, 0)),
    out_shape=jax.ShapeDtypeStruct((N, D), jnp.float32),
)


def _final_body(agg_ref, deg_ref, b_ref, o_ref):
    dis = _dis_of(deg_ref)
    agg = agg_ref[0] + agg_ref[1]
    o_ref[...] = agg * dis[:, None] + b_ref[...]


_final_call = pl.pallas_call(
    _final_body,
    grid=GRID,
    in_specs=[
        pl.BlockSpec((NC, RB, D), lambda i: (0, i, 0)),
        pl.BlockSpec((NC, RB), lambda i: (0, i)),
        pl.BlockSpec((1, D), lambda i: (0, 0)),
    ],
    out_specs=pl.BlockSpec((RB, D), lambda i: (i, 0)),
    out_shape=jax.ShapeDtypeStruct((N, D), jnp.float32),
)


@jax.jit
def kernel(x, edgeIndex, W1, b1, W2, b2):
    P = E_PAD - E
    pad_rows = jnp.arange(P, dtype=jnp.int32) % N
    pad_cols = N + jnp.arange(P, dtype=jnp.int32) % (NPAD - N)
    rows = jnp.concatenate([edgeIndex[0], pad_rows]).reshape(NC, NS, CHUNKS, K)
    cols = jnp.concatenate([edgeIndex[1], pad_cols]).reshape(NC, NS, CHUNKS, K)
    deg2 = _deg_call(cols)
    xw1 = _mm1_call(x, W1, deg2)
    agg1 = _agg_call(xw1, rows, cols)
    xw2 = _mm2_call(agg1, deg2, W2, b1.reshape(1, D))
    agg2 = _agg_call(xw2, rows, cols)
    return _final_call(agg2, deg2, b2.reshape(1, D))


# speedup vs baseline: 1.0639x; 1.0037x over previous
import jax
import jax.numpy as jnp
from jax import lax
from jax.experimental import pallas as pl
from jax.experimental.pallas import tpu as pltpu
from jax.experimental.pallas import tpu_sc as plsc

N = 10000
E = 320000
D = 128
NC = 2
NS = 16
NW = NC * NS
K = 64
CHUNKS = 160
EPT = CHUNKS * K
E_PAD = EPT * NW
NPAD = 640 * NS
RPT = NPAD // NS

_MESH = plsc.VectorSubcoreMesh(core_axis_name="c", subcore_axis_name="s")


def _fill_1d(ref, n, value):
    v = jnp.full((16,), value, jnp.float32)

    def body(i, carry):
        ref[pl.ds(i * 16, 16)] = v
        return carry

    lax.fori_loop(0, n // 16, body, 0)


def _fill_2d(ref, nrows, ncols, value):
    v = jnp.full((16,), value, jnp.float32)
    nv = ncols // 16

    def body(i, carry):
        ref[i // nv, pl.ds((i % nv) * 16, 16)] = v
        return carry

    lax.fori_loop(0, nrows * nv, body, 0)


DEG_DEPTH = 8
DEG_K = 128
DEG_CHUNKS = EPT // DEG_K


def _deg_body(cols_hbm, deg_out, cols_v, ones_v, zb_v, acc_sh, dsem):
    c = lax.axis_index("c")
    s = lax.axis_index("s")
    _fill_1d(ones_v, DEG_K, 1.0)
    _fill_1d(zb_v, 640, 0.0)
    pltpu.sync_copy(zb_v, acc_sh.at[pl.ds(s * 640, 640)])
    plsc.subcore_barrier()
    pltpu.sync_copy(cols_hbm.at[c, s], cols_v)

    def issue(j):
        pltpu.async_copy(ones_v, acc_sh.at[cols_v.at[j]], dsem, add=True)

    def wait_d():
        pltpu.make_async_copy(deg_out.at[0, pl.ds(0, DEG_K)],
                              ones_v, dsem).wait()

    for j in range(DEG_DEPTH):
        issue(j)

    def body(j, carry):
        wait_d()
        issue(j + DEG_DEPTH)
        return carry

    lax.fori_loop(0, DEG_CHUNKS - DEG_DEPTH, body, 0)
    for _ in range(DEG_DEPTH):
        wait_d()
    plsc.subcore_barrier()
    pltpu.sync_copy(acc_sh.at[pl.ds(s * 640, 640)],
                    deg_out.at[c, pl.ds(s * 640, 640)])


_deg_call = pl.kernel(
    _deg_body,
    out_type=jax.ShapeDtypeStruct((NC, NPAD), jnp.float32),
    mesh=_MESH,
    scratch_types=[
        pltpu.VMEM((DEG_CHUNKS, DEG_K), jnp.int32),
        pltpu.VMEM((DEG_K,), jnp.float32),
        pltpu.VMEM((640,), jnp.float32),
        pltpu.VMEM_SHARED((NPAD,), jnp.float32),
        pltpu.SemaphoreType.DMA,
    ],
)


SPAN = 32
NSPANS = CHUNKS // SPAN


def _agg_body(xw_hbm, rows_hbm, cols_hbm, out_hbm,
              idx_v, gbuf, acc_sh, gsem, ssem):
    c = lax.axis_index("c")
    s = lax.axis_index("s")

    def stage(off, ib):
        pltpu.sync_copy(rows_hbm.at[c, s, pl.ds(off, SPAN)],
                        idx_v.at[ib, pl.ds(0, SPAN)])
        pltpu.sync_copy(cols_hbm.at[c, s, pl.ds(off, SPAN)],
                        idx_v.at[ib, pl.ds(SPAN, SPAN)])

    def gather(t, b):
        ib = lax.rem(lax.div(t, SPAN), 2)
        pltpu.async_copy(xw_hbm.at[idx_v.at[ib, lax.rem(t, SPAN)]],
                         gbuf.at[b], gsem)

    def scatter(t, b):
        ib = lax.rem(lax.div(t, SPAN), 2)
        pltpu.async_copy(gbuf.at[b],
                         acc_sh.at[idx_v.at[ib, SPAN + lax.rem(t, SPAN)]],
                         ssem, add=True)

    def wait_g():
        pltpu.make_async_copy(xw_hbm.at[pl.ds(0, K)], gbuf.at[0], gsem).wait()

    def wait_s():
        pltpu.make_async_copy(xw_hbm.at[pl.ds(0, K)], gbuf.at[0], ssem).wait()

    n = CHUNKS
    stage(0, 0)
    gather(0, 0)
    gather(1, 1)
    _fill_2d(gbuf.at[2], K, D, 0.0)
    for t in range(RPT // K):
        pltpu.sync_copy(gbuf.at[2], acc_sh.at[pl.ds(s * RPT + t * K, K)])
    plsc.subcore_barrier()
    wait_g()
    scatter(0, 0)
    gather(2, 2)
    wait_g()
    scatter(1, 1)

    def body(t, carry):
        @pl.when(jnp.logical_and(lax.rem(t, SPAN) == 4,
                                 t < (NSPANS - 1) * SPAN))
        def _():
            sp_next = lax.div(t, SPAN) + 1
            stage(sp_next * SPAN, lax.rem(sp_next, 2))

        wait_s()
        gather(t + 1, lax.rem(t + 1, 3))
        wait_g()
        scatter(t, lax.rem(t, 3))
        return carry

    lax.fori_loop(2, n - 1, body, 0)
    wait_s()
    wait_g()
    scatter(n - 1, (n - 1) % 3)
    wait_s()
    wait_s()
    plsc.subcore_barrier()
    pltpu.sync_copy(acc_sh.at[pl.ds(s * RPT, RPT)],
                    out_hbm.at[c, pl.ds(s * RPT, RPT)])


_agg_call = pl.kernel(
    _agg_body,
    out_type=jax.ShapeDtypeStruct((NC, NPAD, D), jnp.float32),
    mesh=_MESH,
    scratch_types=[
        pltpu.VMEM((2, 2 * SPAN, K), jnp.int32),
        pltpu.VMEM((3, K, D), jnp.float32),
        pltpu.VMEM_SHARED((NPAD, D), jnp.float32),
        pltpu.SemaphoreType.DMA,
        pltpu.SemaphoreType.DMA,
    ],
)


RB = 2048
GRID = (NPAD // RB,)


def _dis_of(deg_ref):
    deg = deg_ref[0, :] + deg_ref[1, :]
    return jnp.where(deg > 0, lax.rsqrt(deg), 0.0)


def _mm1_body(x_ref, w_ref, deg_ref, o_ref):
    dis = _dis_of(deg_ref)
    xw = jnp.dot(x_ref[...], w_ref[...],
                 preferred_element_type=jnp.float32,
                 precision=lax.Precision.HIGHEST)
    o_ref[...] = xw * dis[:, None]


_mm1_call = pl.pallas_call(
    _mm1_body,
    grid=GRID,
    in_specs=[
        pl.BlockSpec((RB, D), lambda i: (i, 0)),
        pl.BlockSpec((D, D), lambda i: (0, 0)),
        pl.BlockSpec((NC, RB), lambda i: (0, i)),
    ],
    out_specs=pl.BlockSpec((RB, D), lambda i: (i, 0)),
    out_shape=jax.ShapeDtypeStruct((N, D), jnp.float32),
)


def _mm2_body(agg_ref, deg_ref, w_ref, b_ref, o_ref):
    dis = _dis_of(deg_ref)
    agg = agg_ref[0] + agg_ref[1]
    h = jnp.maximum(agg * dis[:, None] + b_ref[...], 0.0)
    hw = jnp.dot(h, w_ref[...],
                 preferred_element_type=jnp.float32,
                 precision=lax.Precision.HIGHEST)
    o_ref[...] = hw * dis[:, None]


_mm2_call = pl.pallas_call(
    _mm2_body,
    grid=GRID,
    in_specs=[
        pl.BlockSpec((NC, RB, D), lambda i: (0, i, 0)),
        pl.BlockSpec((NC, RB), lambda i: (0, i)),
        pl.BlockSpec((D, D), lambda i: (0, 0)),
        pl.BlockSpec((1, D), lambda i: (0, 0)),
    ],
    out_specs=pl.BlockSpec((RB, D), lambda i: (i, 0)),
    out_shape=jax.ShapeDtypeStruct((N, D), jnp.float32),
)


def _final_body(agg_ref, deg_ref, b_ref, o_ref):
    dis = _dis_of(deg_ref)
    agg = agg_ref[0] + agg_ref[1]
    o_ref[...] = agg * dis[:, None] + b_ref[...]


_final_call = pl.pallas_call(
    _final_body,
    grid=GRID,
    in_specs=[
        pl.BlockSpec((NC, RB, D), lambda i: (0, i, 0)),
        pl.BlockSpec((NC, RB), lambda i: (0, i)),
        pl.BlockSpec((1, D), lambda i: (0, 0)),
    ],
    out_specs=pl.BlockSpec((RB, D), lambda i: (i, 0)),
    out_shape=jax.ShapeDtypeStruct((N, D), jnp.float32),
)


@jax.jit
def kernel(x, edgeIndex, W1, b1, W2, b2):
    P = E_PAD - E
    pad_rows = jnp.arange(P, dtype=jnp.int32) % N
    pad_cols = N + jnp.arange(P, dtype=jnp.int32) % (NPAD - N)
    rows = jnp.concatenate([edgeIndex[0], pad_rows]).reshape(NC, NS, CHUNKS, K)
    cols = jnp.concatenate([edgeIndex[1], pad_cols]).reshape(NC, NS, CHUNKS, K)
    deg2 = _deg_call(cols.reshape(NC, NS, DEG_CHUNKS, DEG_K))
    xw1 = _mm1_call(x, W1, deg2)
    agg1 = _agg_call(xw1, rows, cols)
    xw2 = _mm2_call(agg1, deg2, W2, b1.reshape(1, D))
    agg2 = _agg_call(xw2, rows, cols)
    return _final_call(agg2, deg2, b2.reshape(1, D))
